# baseline (speedup 1.0000x reference)
import jax
import jax.numpy as jnp
from jax.experimental import pallas as pl

_MARGIN = 0.3
_S = 15.0
_B = 1024
_C = 100000
_RB = 8
_LOG2E = 1.4426950408889634
_LN2 = 0.6931471805599453


def _body(costh_ref, lab_ref, out_ref):
    i = pl.program_id(0)
    x = costh_ref[...]
    lab = lab_ref[...]
    y = x * (_S * _LOG2E)
    cols = jax.lax.broadcasted_iota(jnp.int32, (_RB, _C), 1)
    is_lab = cols == lab
    yl = jnp.sum(jnp.where(is_lab, y, 0.0), axis=1)
    m = jnp.max(y, axis=1)
    s = jnp.sum(jnp.exp2(y - m[:, None]), axis=1)
    d = _S * _MARGIN * _LOG2E
    s_corr = s - jnp.exp2(yl - m) + jnp.exp2(yl - d - m)
    logz = m + jnp.log2(s_corr)
    part = (_LN2 * jnp.sum(logz - (yl - d))).reshape(1, 1)

    @pl.when(i == 0)
    def _init():
        out_ref[...] = jnp.zeros((1, 1), jnp.float32)

    out_ref[...] += part


def kernel(costh, label):
    lab2d = label.reshape(_B, 1).astype(jnp.int32)
    total = pl.pallas_call(
        _body,
        grid=(_B // _RB,),
        in_specs=[
            pl.BlockSpec((_RB, _C), lambda i: (i, 0)),
            pl.BlockSpec((_RB, 1), lambda i: (i, 0)),
        ],
        out_specs=pl.BlockSpec((1, 1), lambda i: (0, 0)),
        out_shape=jax.ShapeDtypeStruct((1, 1), jnp.float32),
    )(costh, lab2d)
    return total[0, 0] / _B


# speedup vs baseline: 1.2014x; 1.2014x over previous
import jax
import jax.numpy as jnp
from jax.experimental import pallas as pl

_MARGIN = 0.3
_S = 15.0
_B = 1024
_C = 100000
_RB = 8
_LOG2E = 1.4426950408889634
_LN2 = 0.6931471805599453


def _body(costh_ref, lab_ref, out_ref):
    i = pl.program_id(0)
    x = costh_ref[...]
    lab = lab_ref[...]
    y = x * (_S * _LOG2E)
    cols = jax.lax.broadcasted_iota(jnp.int32, (_RB, _C), 1)
    is_lab = cols == lab
    yl = jnp.sum(jnp.where(is_lab, y, 0.0), axis=1)
    s = jnp.sum(jnp.exp2(y), axis=1)
    d = _S * _MARGIN * _LOG2E
    s_corr = s - jnp.exp2(yl) + jnp.exp2(yl - d)
    logz = jnp.log2(s_corr)
    part = (_LN2 * jnp.sum(logz - (yl - d))).reshape(1, 1)

    @pl.when(i == 0)
    def _init():
        out_ref[...] = jnp.zeros((1, 1), jnp.float32)

    out_ref[...] += part


def kernel(costh, label):
    lab2d = label.reshape(_B, 1).astype(jnp.int32)
    total = pl.pallas_call(
        _body,
        grid=(_B // _RB,),
        in_specs=[
            pl.BlockSpec((_RB, _C), lambda i: (i, 0)),
            pl.BlockSpec((_RB, 1), lambda i: (i, 0)),
        ],
        out_specs=pl.BlockSpec((1, 1), lambda i: (0, 0)),
        out_shape=jax.ShapeDtypeStruct((1, 1), jnp.float32),
    )(costh, lab2d)
    return total[0, 0] / _B
